# baseline (speedup 1.0000x reference)
import functools
import math

import jax
import jax.numpy as jnp
from jax import lax
from jax.experimental import pallas as pl
from jax.experimental.pallas import tpu as pltpu
from jax.experimental.pallas import tpu_sc as plsc

HID = 128
NT = 9
TS = [3, 1, 3, 1]
RES = {1: [0], 3: [0, 1]}
BLK = 1024
NC, NS = 2, 16
NW = NC * NS
GCH = 128


def _sc_gather(h2, idx3, cap):
    nch = idx3.shape[1]
    bpw = nch * GCH
    mesh = plsc.VectorSubcoreMesh(core_axis_name="c", subcore_axis_name="s")

    @functools.partial(
        pl.kernel,
        out_type=jax.ShapeDtypeStruct((cap, HID), jnp.float32),
        mesh=mesh,
        scratch_types=[
            pltpu.VMEM((nch, GCH), jnp.int32),
            pltpu.VMEM((2, GCH, HID), jnp.float32),
            pltpu.SemaphoreType.DMA,
            pltpu.SemaphoreType.DMA,
        ],
    )
    def k(h_hbm, idx_hbm, out_hbm, idx_v, buf, gsem, ssem):
        del ssem
        wid = lax.axis_index("s") * NC + lax.axis_index("c")
        base = wid * bpw
        pltpu.sync_copy(idx_hbm.at[wid], idx_v)

        def body(j, carry):
            pltpu.async_copy(h_hbm.at[idx_v.at[j]], buf.at[0], gsem).wait()
            pltpu.sync_copy(buf.at[0], out_hbm.at[pl.ds(base + j * GCH, GCH)])
            return carry

        lax.fori_loop(0, nch, body, 0)

    return k(h2, idx3)


def _segmm_body(x_ref, w_ref, b_ref, o_ref):
    o_ref[...] = (
        jnp.dot(x_ref[...], w_ref[0], preferred_element_type=jnp.float32)
        + b_ref[0]
    )


def _segmm(gathered, wblk, bblk):
    cap = gathered.shape[0]
    nb = cap // BLK
    return pl.pallas_call(
        _segmm_body,
        grid=(nb,),
        in_specs=[
            pl.BlockSpec((BLK, HID), lambda j: (j, 0)),
            pl.BlockSpec((1, HID, HID), lambda j: (j, 0, 0)),
            pl.BlockSpec((1, 1, HID), lambda j: (j, 0, 0)),
        ],
        out_specs=pl.BlockSpec((BLK, HID), lambda j: (j, 0)),
        out_shape=jax.ShapeDtypeStruct((cap, HID), jnp.float32),
    )(gathered, wblk, bblk)


def _gru_body(x_ref, h_ref, k_ref, rk_ref, b0_ref, b1_ref, o_ref):
    mx = (
        jnp.dot(x_ref[...], k_ref[...], preferred_element_type=jnp.float32)
        + b0_ref[...]
    )
    mh = (
        jnp.dot(h_ref[...], rk_ref[...], preferred_element_type=jnp.float32)
        + b1_ref[...]
    )
    h = h_ref[...]
    z = jax.nn.sigmoid(mx[:, :HID] + mh[:, :HID])
    r = jax.nn.sigmoid(mx[:, HID:2 * HID] + mh[:, HID:2 * HID])
    hh = jnp.tanh(mx[:, 2 * HID:] + r * mh[:, 2 * HID:])
    o_ref[...] = z * h + (1.0 - z) * hh


def _gru(xcat, h, k, rk, b0, b1, rb=2000):
    r, d = xcat.shape
    return pl.pallas_call(
        _gru_body,
        grid=(r // rb,),
        in_specs=[
            pl.BlockSpec((rb, d), lambda i: (i, 0)),
            pl.BlockSpec((rb, HID), lambda i: (i, 0)),
            pl.BlockSpec((d, 3 * HID), lambda i: (0, 0)),
            pl.BlockSpec((HID, 3 * HID), lambda i: (0, 0)),
            pl.BlockSpec((1, 3 * HID), lambda i: (0, 0)),
            pl.BlockSpec((1, 3 * HID), lambda i: (0, 0)),
        ],
        out_specs=pl.BlockSpec((rb, HID), lambda i: (i, 0)),
        out_shape=jax.ShapeDtypeStruct((r, HID), jnp.float32),
    )(xcat, h, k, rk, b0, b1)


def kernel(states, edge_ids, Wt, bt, gru_k0, gru_rk0, gru_b0, gru_k1, gru_rk1,
           gru_b1, gru_k2, gru_rk2, gru_b2, gru_k3, gru_rk3, gru_b3):
    gk = [gru_k0, gru_k1, gru_k2, gru_k3]
    grk = [gru_rk0, gru_rk1, gru_rk2, gru_rk3]
    gb = [gru_b0, gru_b1, gru_b2, gru_b3]
    b, n, h_dim = states.shape
    bn = b * n
    e = edge_ids.shape[0]
    etype = edge_ids[:, 0]
    eb = edge_ids[:, 1]
    es = edge_ids[:, 2]
    ed = edge_ids[:, 3]

    lcm = BLK * 4096 // math.gcd(BLK, 4096)
    cap = ((e + NT * (BLK - 1) + lcm - 1) // lcm) * lcm
    tgrid = jnp.arange(NT, dtype=jnp.int32)
    onehot = (etype[None, :] == tgrid[:, None]).astype(jnp.int32)
    occ = jnp.cumsum(onehot, axis=1)
    cnts = occ[:, -1]
    pc = ((cnts + BLK - 1) // BLK) * BLK
    pstart = jnp.concatenate(
        [jnp.zeros((1,), jnp.int32), jnp.cumsum(pc).astype(jnp.int32)]
    )
    pos = jnp.sum(onehot * (pstart[:NT, None] + occ - 1), axis=0)
    eid_pad = jnp.full((cap,), e, jnp.int32).at[pos].set(
        jnp.arange(e, dtype=jnp.int32)
    )
    gsrc_pad = jnp.concatenate([eb * n + es, jnp.zeros((1,), jnp.int32)])[eid_pad]
    gdst_pad = jnp.concatenate([eb * n + ed, jnp.full((1,), bn, jnp.int32)])[eid_pad]
    gsrc3 = gsrc_pad.reshape(NW, cap // (NW * GCH), GCH)
    offs = jnp.arange(cap // BLK, dtype=jnp.int32) * BLK
    tmap = jnp.clip(
        jnp.searchsorted(pstart, offs, side="right") - 1, 0, NT - 1
    ).astype(jnp.int32)

    layer_states = [states.reshape(bn, h_dim)]
    for l, steps in enumerate(TS):
        k, rk = gk[l], grk[l]
        b0, b1 = gb[l][0:1], gb[l][1:2]
        wblk = Wt[l][tmap]
        bblk = bt[l][tmap][:, None, :]
        for s in range(steps):
            h = layer_states[-1]
            gathered = _sc_gather(h, gsrc3, cap)
            m = _segmm(gathered, wblk, bblk)
            msgs = jnp.zeros((bn + 8, h_dim), jnp.float32).at[gdst_pad].add(m)[:bn]
            parts = [layer_states[ix] for ix in RES.get(l, [])] + [msgs]
            xcat = jnp.concatenate(parts, axis=1) if len(parts) > 1 else msgs
            new = _gru(xcat, h, k, rk, b0, b1)
            if s == 0:
                layer_states.append(new)
            else:
                layer_states[-1] = new
    return layer_states[-1].reshape(b, n, h_dim)


# speedup vs baseline: 1.0058x; 1.0058x over previous
import functools
import math

import jax
import jax.numpy as jnp
from jax import lax
from jax.experimental import pallas as pl
from jax.experimental.pallas import tpu as pltpu
from jax.experimental.pallas import tpu_sc as plsc

HID = 128
NT = 9
TS = [3, 1, 3, 1]
RES = {1: [0], 3: [0, 1]}
BLK = 1024
NC, NS = 2, 16
NW = NC * NS
GCH = 128


def _sc_gather(h2, idx3, cap):
    nch = idx3.shape[1]
    bpw = nch * GCH
    mesh = plsc.VectorSubcoreMesh(core_axis_name="c", subcore_axis_name="s")

    kb = max(kk for kk in range(1, 8) if nch % kk == 0)
    nsup = nch // kb

    @functools.partial(
        pl.kernel,
        out_type=jax.ShapeDtypeStruct((cap, HID), jnp.float32),
        mesh=mesh,
        scratch_types=[
            pltpu.VMEM((nch, GCH), jnp.int32),
            pltpu.VMEM((kb, GCH, HID), jnp.float32),
            pltpu.SemaphoreType.DMA,
            pltpu.SemaphoreType.DMA,
        ],
    )
    def k(h_hbm, idx_hbm, out_hbm, idx_v, buf, gsem, ssem):
        wid = lax.axis_index("s") * NC + lax.axis_index("c")
        base = wid * bpw
        pltpu.sync_copy(idx_hbm.at[wid], idx_v)

        def body(j, carry):
            for t in range(kb):
                pltpu.async_copy(h_hbm.at[idx_v.at[j * kb + t]], buf.at[t], gsem)
            pltpu.make_async_copy(h_hbm.at[idx_v.at[0]], buf, gsem).wait()
            for t in range(kb):
                pltpu.async_copy(
                    buf.at[t],
                    out_hbm.at[pl.ds(base + (j * kb + t) * GCH, GCH)],
                    ssem,
                )
            pltpu.make_async_copy(
                buf, out_hbm.at[pl.ds(base, kb * GCH)], ssem
            ).wait()
            return carry

        lax.fori_loop(0, nsup, body, 0)

    return k(h2, idx3)


def _segmm_body(x_ref, w_ref, b_ref, o_ref):
    o_ref[...] = (
        jnp.dot(x_ref[...], w_ref[0], preferred_element_type=jnp.float32)
        + b_ref[0]
    )


def _segmm(gathered, wblk, bblk):
    cap = gathered.shape[0]
    nb = cap // BLK
    return pl.pallas_call(
        _segmm_body,
        grid=(nb,),
        in_specs=[
            pl.BlockSpec((BLK, HID), lambda j: (j, 0)),
            pl.BlockSpec((1, HID, HID), lambda j: (j, 0, 0)),
            pl.BlockSpec((1, 1, HID), lambda j: (j, 0, 0)),
        ],
        out_specs=pl.BlockSpec((BLK, HID), lambda j: (j, 0)),
        out_shape=jax.ShapeDtypeStruct((cap, HID), jnp.float32),
    )(gathered, wblk, bblk)


def _gru_body(x_ref, h_ref, k_ref, rk_ref, b0_ref, b1_ref, o_ref):
    mx = (
        jnp.dot(x_ref[...], k_ref[...], preferred_element_type=jnp.float32)
        + b0_ref[...]
    )
    mh = (
        jnp.dot(h_ref[...], rk_ref[...], preferred_element_type=jnp.float32)
        + b1_ref[...]
    )
    h = h_ref[...]
    z = jax.nn.sigmoid(mx[:, :HID] + mh[:, :HID])
    r = jax.nn.sigmoid(mx[:, HID:2 * HID] + mh[:, HID:2 * HID])
    hh = jnp.tanh(mx[:, 2 * HID:] + r * mh[:, 2 * HID:])
    o_ref[...] = z * h + (1.0 - z) * hh


def _gru(xcat, h, k, rk, b0, b1, rb=2000):
    r, d = xcat.shape
    return pl.pallas_call(
        _gru_body,
        grid=(r // rb,),
        in_specs=[
            pl.BlockSpec((rb, d), lambda i: (i, 0)),
            pl.BlockSpec((rb, HID), lambda i: (i, 0)),
            pl.BlockSpec((d, 3 * HID), lambda i: (0, 0)),
            pl.BlockSpec((HID, 3 * HID), lambda i: (0, 0)),
            pl.BlockSpec((1, 3 * HID), lambda i: (0, 0)),
            pl.BlockSpec((1, 3 * HID), lambda i: (0, 0)),
        ],
        out_specs=pl.BlockSpec((rb, HID), lambda i: (i, 0)),
        out_shape=jax.ShapeDtypeStruct((r, HID), jnp.float32),
    )(xcat, h, k, rk, b0, b1)


def kernel(states, edge_ids, Wt, bt, gru_k0, gru_rk0, gru_b0, gru_k1, gru_rk1,
           gru_b1, gru_k2, gru_rk2, gru_b2, gru_k3, gru_rk3, gru_b3):
    gk = [gru_k0, gru_k1, gru_k2, gru_k3]
    grk = [gru_rk0, gru_rk1, gru_rk2, gru_rk3]
    gb = [gru_b0, gru_b1, gru_b2, gru_b3]
    b, n, h_dim = states.shape
    bn = b * n
    e = edge_ids.shape[0]
    etype = edge_ids[:, 0]
    eb = edge_ids[:, 1]
    es = edge_ids[:, 2]
    ed = edge_ids[:, 3]

    lcm = BLK * 4096 // math.gcd(BLK, 4096)
    cap = ((e + NT * (BLK - 1) + lcm - 1) // lcm) * lcm
    tgrid = jnp.arange(NT, dtype=jnp.int32)
    onehot = (etype[None, :] == tgrid[:, None]).astype(jnp.int32)
    occ = jnp.cumsum(onehot, axis=1)
    cnts = occ[:, -1]
    pc = ((cnts + BLK - 1) // BLK) * BLK
    pstart = jnp.concatenate(
        [jnp.zeros((1,), jnp.int32), jnp.cumsum(pc).astype(jnp.int32)]
    )
    pos = jnp.sum(onehot * (pstart[:NT, None] + occ - 1), axis=0)
    eid_pad = jnp.full((cap,), e, jnp.int32).at[pos].set(
        jnp.arange(e, dtype=jnp.int32)
    )
    gsrc_pad = jnp.concatenate([eb * n + es, jnp.zeros((1,), jnp.int32)])[eid_pad]
    gdst_pad = jnp.concatenate([eb * n + ed, jnp.full((1,), bn, jnp.int32)])[eid_pad]
    gsrc3 = gsrc_pad.reshape(NW, cap // (NW * GCH), GCH)
    offs = jnp.arange(cap // BLK, dtype=jnp.int32) * BLK
    tmap = jnp.clip(
        jnp.searchsorted(pstart, offs, side="right") - 1, 0, NT - 1
    ).astype(jnp.int32)

    layer_states = [states.reshape(bn, h_dim)]
    for l, steps in enumerate(TS):
        k, rk = gk[l], grk[l]
        b0, b1 = gb[l][0:1], gb[l][1:2]
        wblk = Wt[l][tmap]
        bblk = bt[l][tmap][:, None, :]
        for s in range(steps):
            h = layer_states[-1]
            gathered = _sc_gather(h, gsrc3, cap)
            m = _segmm(gathered, wblk, bblk)
            msgs = jnp.zeros((bn + 8, h_dim), jnp.float32).at[gdst_pad].add(m)[:bn]
            parts = [layer_states[ix] for ix in RES.get(l, [])] + [msgs]
            xcat = jnp.concatenate(parts, axis=1) if len(parts) > 1 else msgs
            new = _gru(xcat, h, k, rk, b0, b1)
            if s == 0:
                layer_states.append(new)
            else:
                layer_states[-1] = new
    return layer_states[-1].reshape(b, n, h_dim)


# speedup vs baseline: 1.4387x; 1.4304x over previous
import functools
import math

import jax
import jax.numpy as jnp
from jax import lax
from jax.experimental import pallas as pl
from jax.experimental.pallas import tpu as pltpu
from jax.experimental.pallas import tpu_sc as plsc

HID = 128
NT = 9
TS = [3, 1, 3, 1]
RES = {1: [0], 3: [0, 1]}
BLK = 1024
NC, NS = 2, 16
NW = NC * NS
GCH = 128


def _sc_gather(h2, idx3, cap):
    nch = idx3.shape[1]
    bpw = nch * GCH
    mesh = plsc.VectorSubcoreMesh(core_axis_name="c", subcore_axis_name="s")

    kb = max(kk for kk in range(1, 8) if nch % kk == 0)
    nsup = nch // kb

    @functools.partial(
        pl.kernel,
        out_type=jax.ShapeDtypeStruct((cap, HID), jnp.float32),
        mesh=mesh,
        scratch_types=[
            pltpu.VMEM((nch, GCH), jnp.int32),
            pltpu.VMEM((kb, GCH, HID), jnp.float32),
            pltpu.SemaphoreType.DMA,
            pltpu.SemaphoreType.DMA,
        ],
    )
    def k(h_hbm, idx_hbm, out_hbm, idx_v, buf, gsem, ssem):
        wid = lax.axis_index("s") * NC + lax.axis_index("c")
        base = wid * bpw
        pltpu.sync_copy(idx_hbm.at[wid], idx_v)

        def body(j, carry):
            for t in range(kb):
                pltpu.async_copy(h_hbm.at[idx_v.at[j * kb + t]], buf.at[t], gsem)
            pltpu.make_async_copy(h_hbm.at[idx_v.at[0]], buf, gsem).wait()
            for t in range(kb):
                pltpu.async_copy(
                    buf.at[t],
                    out_hbm.at[pl.ds(base + (j * kb + t) * GCH, GCH)],
                    ssem,
                )
            pltpu.make_async_copy(
                buf, out_hbm.at[pl.ds(base, kb * GCH)], ssem
            ).wait()
            return carry

        lax.fori_loop(0, nsup, body, 0)

    return k(h2, idx3)


def _proj_body(x_ref, w_ref, b_ref, o_ref):
    o_ref[...] = (
        jnp.dot(x_ref[...], w_ref[...], preferred_element_type=jnp.float32)
        + b_ref[...]
    )


def _proj(h2, wall, bias, rb=2000):
    r = h2.shape[0]
    return pl.pallas_call(
        _proj_body,
        grid=(r // rb,),
        in_specs=[
            pl.BlockSpec((rb, HID), lambda i: (i, 0)),
            pl.BlockSpec((HID, NT * HID), lambda i: (0, 0)),
            pl.BlockSpec((1, NT * HID), lambda i: (0, 0)),
        ],
        out_specs=pl.BlockSpec((rb, NT * HID), lambda i: (i, 0)),
        out_shape=jax.ShapeDtypeStruct((r, NT * HID), jnp.float32),
    )(h2, wall, bias)


def _gru_body(x_ref, h_ref, k_ref, rk_ref, b0_ref, b1_ref, o_ref):
    mx = (
        jnp.dot(x_ref[...], k_ref[...], preferred_element_type=jnp.float32)
        + b0_ref[...]
    )
    mh = (
        jnp.dot(h_ref[...], rk_ref[...], preferred_element_type=jnp.float32)
        + b1_ref[...]
    )
    h = h_ref[...]
    z = jax.nn.sigmoid(mx[:, :HID] + mh[:, :HID])
    r = jax.nn.sigmoid(mx[:, HID:2 * HID] + mh[:, HID:2 * HID])
    hh = jnp.tanh(mx[:, 2 * HID:] + r * mh[:, 2 * HID:])
    o_ref[...] = z * h + (1.0 - z) * hh


def _gru(xcat, h, k, rk, b0, b1, rb=2000):
    r, d = xcat.shape
    return pl.pallas_call(
        _gru_body,
        grid=(r // rb,),
        in_specs=[
            pl.BlockSpec((rb, d), lambda i: (i, 0)),
            pl.BlockSpec((rb, HID), lambda i: (i, 0)),
            pl.BlockSpec((d, 3 * HID), lambda i: (0, 0)),
            pl.BlockSpec((HID, 3 * HID), lambda i: (0, 0)),
            pl.BlockSpec((1, 3 * HID), lambda i: (0, 0)),
            pl.BlockSpec((1, 3 * HID), lambda i: (0, 0)),
        ],
        out_specs=pl.BlockSpec((rb, HID), lambda i: (i, 0)),
        out_shape=jax.ShapeDtypeStruct((r, HID), jnp.float32),
    )(xcat, h, k, rk, b0, b1)


def kernel(states, edge_ids, Wt, bt, gru_k0, gru_rk0, gru_b0, gru_k1, gru_rk1,
           gru_b1, gru_k2, gru_rk2, gru_b2, gru_k3, gru_rk3, gru_b3):
    gk = [gru_k0, gru_k1, gru_k2, gru_k3]
    grk = [gru_rk0, gru_rk1, gru_rk2, gru_rk3]
    gb = [gru_b0, gru_b1, gru_b2, gru_b3]
    b, n, h_dim = states.shape
    bn = b * n
    e = edge_ids.shape[0]
    etype = edge_ids[:, 0]
    eb = edge_ids[:, 1]
    es = edge_ids[:, 2]
    ed = edge_ids[:, 3]

    cap = ((e + NW * GCH - 1) // (NW * GCH)) * (NW * GCH)
    pad = cap - e
    gidx = (eb * n + es) * NT + etype
    gidx_pad = jnp.concatenate([gidx, jnp.zeros((pad,), jnp.int32)])
    gdst_pad = jnp.concatenate([eb * n + ed, jnp.full((pad,), bn, jnp.int32)])
    gidx3 = gidx_pad.reshape(NW, cap // (NW * GCH), GCH)

    layer_states = [states.reshape(bn, h_dim)]
    for l, steps in enumerate(TS):
        k, rk = gk[l], grk[l]
        b0, b1 = gb[l][0:1], gb[l][1:2]
        wall = Wt[l].transpose(1, 0, 2).reshape(h_dim, NT * h_dim)
        bias = bt[l].reshape(1, NT * h_dim)
        for s in range(steps):
            h = layer_states[-1]
            y9 = _proj(h, wall, bias).reshape(bn * NT, h_dim)
            gathered = _sc_gather(y9, gidx3, cap)
            msgs = jnp.zeros((bn + 8, h_dim), jnp.float32).at[gdst_pad].add(
                gathered
            )[:bn]
            parts = [layer_states[ix] for ix in RES.get(l, [])] + [msgs]
            xcat = jnp.concatenate(parts, axis=1) if len(parts) > 1 else msgs
            new = _gru(xcat, h, k, rk, b0, b1)
            if s == 0:
                layer_states.append(new)
            else:
                layer_states[-1] = new
    return layer_states[-1].reshape(b, n, h_dim)
